# baseline (speedup 1.0000x reference)
import jax
import jax.numpy as jnp
from jax.experimental import pallas as pl
from jax.experimental.pallas import tpu as pltpu

_B, _L, _D, _E, _G = 128, 32, 512, 8, 2
_LD = _L * _D
_CK = 2048
_KPE = _LD // _CK
_NW = _E * _KPE
_NG = _G * _KPE
_EPS = 1e-9


def _moe_kernel(logits_ref, mask_ref, xf_ref, w_ref, wg_ref, b_ref, bg_ref,
                out_ref, acc_ref, g_ref):
    i = pl.program_id(0)

    @pl.when(i == 0)
    def _init():
        logits = logits_ref[...]
        mask = (mask_ref[...] == 1).astype(jnp.float32)
        m = jnp.max(logits, axis=1, keepdims=True)
        ex = jnp.exp(logits - m)
        sm = ex / jnp.sum(ex, axis=1, keepdims=True)
        gg = sm * mask
        gg = gg / (jnp.sum(gg, axis=1, keepdims=True) + _EPS)
        g_ref[...] = gg
        acc_ref[...] = jnp.zeros_like(acc_ref)

    @pl.when(i < _NW)
    def _expert_step():
        e = i // _KPE
        k = i % _KPE
        x = xf_ref[:, pl.ds(k * _CK, _CK)]
        part = jnp.dot(x, w_ref[0].astype(jnp.bfloat16),
                       preferred_element_type=jnp.float32)
        lane = jax.lax.broadcasted_iota(jnp.int32, (_B, _E), 1)
        ge = jnp.sum(jnp.where(lane == e, g_ref[...], 0.0), axis=1,
                     keepdims=True)
        acc_ref[...] += ge * part

    @pl.when(i == _NW - 1)
    def _finish_experts():
        gb = jax.lax.dot_general(
            g_ref[...], b_ref[...], (((1,), (0,)), ((), ())),
            preferred_element_type=jnp.float32)
        c = acc_ref[...] + gb
        acc_ref[...] = c.astype(jnp.bfloat16).astype(jnp.float32)

    @pl.when(i >= _NW)
    def _general_step():
        j = i - _NW
        k = jax.lax.rem(j, _KPE)
        x = xf_ref[:, pl.ds(k * _CK, _CK)]
        acc_ref[...] += jnp.dot(x, wg_ref[0].astype(jnp.bfloat16),
                                preferred_element_type=jnp.float32)

    @pl.when(i == _NW + _NG - 1)
    def _finish():
        bgs = jnp.sum(bg_ref[...], axis=0, keepdims=True)
        out_ref[...] = acc_ref[...] + bgs


def kernel(cycle_curve_data, logits, moe_masks, W, b, Wg, bg):
    xf = cycle_curve_data.reshape(_B, _LD).astype(jnp.bfloat16)
    masks = moe_masks.astype(jnp.int32)

    grid = (_NW + _NG,)

    def w_index(i):
        ic = jnp.minimum(i, _NW - 1)
        return ic // _KPE, ic % _KPE, 0

    def wg_index(i):
        j = jnp.clip(i - _NW, 0, _NG - 1)
        return j // _KPE, jax.lax.rem(j, _KPE), 0

    out = pl.pallas_call(
        _moe_kernel,
        grid=grid,
        in_specs=[
            pl.BlockSpec((_B, _E), lambda i: (0, 0)),
            pl.BlockSpec((_B, _E), lambda i: (0, 0)),
            pl.BlockSpec((_B, _LD), lambda i: (0, 0)),
            pl.BlockSpec((1, _CK, _D), w_index),
            pl.BlockSpec((1, _CK, _D), wg_index),
            pl.BlockSpec((_E, _D), lambda i: (0, 0)),
            pl.BlockSpec((_G, _D), lambda i: (0, 0)),
        ],
        out_specs=pl.BlockSpec((_B, _D), lambda i: (0, 0)),
        out_shape=jax.ShapeDtypeStruct((_B, _D), jnp.float32),
        scratch_shapes=[
            pltpu.VMEM((_B, _D), jnp.float32),
            pltpu.VMEM((_B, _E), jnp.float32),
        ],
        compiler_params=pltpu.CompilerParams(
            dimension_semantics=("arbitrary",),
        ),
    )(logits, masks, xf, W, Wg, b, bg)
    return out
